# baseline (speedup 1.0000x reference)
import functools

import jax
import jax.numpy as jnp
from jax import lax
from jax.experimental import pallas as pl
from jax.experimental.pallas import tpu as pltpu
from jax.experimental.pallas import tpu_sc as plsc

_NBUF = 8


def _emb_lookup(weight, ids):
    B, H = ids.shape
    _, D = weight.shape
    info = plsc.get_sparse_core_info()
    num_cores = info.num_cores
    nw = num_cores * info.num_subcores
    bpw = B // nw
    assert bpw % _NBUF == 0
    rounds = bpw // _NBUF

    mesh = plsc.VectorSubcoreMesh(core_axis_name="c", subcore_axis_name="s")

    @functools.partial(
        pl.kernel,
        mesh=mesh,
        compiler_params=pltpu.CompilerParams(
            use_tc_tiling_on_sc=False, needs_layout_passes=False),
        out_type=jax.ShapeDtypeStruct((B, H, D), jnp.float32),
        scratch_types=[
            [pltpu.VMEM((H,), jnp.int32) for _ in range(_NBUF)],
            [pltpu.VMEM((H, D), jnp.float32) for _ in range(_NBUF)],
            [pltpu.SemaphoreType.DMA for _ in range(_NBUF)],
            [pltpu.SemaphoreType.DMA for _ in range(_NBUF)],
        ],
    )
    def emb(w_hbm, idx_hbm, out_hbm, idx_v, rows_v, gsem, ssem):
        wid = lax.axis_index("s") * num_cores + lax.axis_index("c")
        b0 = wid * bpw

        def fire(g, ib):
            pltpu.sync_copy(idx_hbm.at[b0 + g, :], idx_v[ib])
            pltpu.async_copy(w_hbm.at[idx_v[ib]], rows_v[ib], gsem[ib])

        def wait_gather(ib):
            pltpu.make_async_copy(
                w_hbm.at[idx_v[ib]], rows_v[ib], gsem[ib]).wait()

        def store(g, ib):
            pltpu.async_copy(rows_v[ib], out_hbm.at[b0 + g], ssem[ib])

        def wait_store(ib):
            pltpu.make_async_copy(
                rows_v[ib], out_hbm.at[b0], ssem[ib]).wait()

        for b in range(_NBUF - 1):
            fire(b, b)

        def body(r, carry):
            for b in range(_NBUF):
                g = r * _NBUF + b
                wait_gather(b)
                store(g, b)
                ibf = (b - 1) % _NBUF
                gf = g + _NBUF - 1

                @pl.when(gf < bpw)
                def _():
                    @pl.when(gf >= _NBUF)
                    def _():
                        wait_store(ibf)

                    fire(gf, ibf)

            return carry

        lax.fori_loop(0, rounds, body, 0)
        for b in range(_NBUF):
            wait_store(b)

    return emb(weight, ids)


def kernel(token_ids, weight):
    return _emb_lookup(weight, token_ids.astype(jnp.int32))


# speedup vs baseline: 1.0298x; 1.0298x over previous
import functools

import jax
import jax.numpy as jnp
from jax import lax
from jax.experimental import pallas as pl
from jax.experimental.pallas import tpu as pltpu
from jax.experimental.pallas import tpu_sc as plsc

_NBUF = 4
_W = 800


def _emb_lookup(weight, ids_flat):
    T, = ids_flat.shape
    _, D = weight.shape
    info = plsc.get_sparse_core_info()
    num_cores = info.num_cores
    nw = num_cores * info.num_subcores
    tpw = T // nw
    wins = tpw // _W
    assert tpw % _W == 0 and wins % _NBUF == 0
    rounds = wins // _NBUF

    mesh = plsc.VectorSubcoreMesh(core_axis_name="c", subcore_axis_name="s")

    @functools.partial(
        pl.kernel,
        mesh=mesh,
        compiler_params=pltpu.CompilerParams(
            use_tc_tiling_on_sc=False, needs_layout_passes=False),
        out_type=jax.ShapeDtypeStruct((T, D), jnp.float32),
        scratch_types=[
            [pltpu.VMEM((_W,), jnp.int32) for _ in range(_NBUF)],
            [pltpu.VMEM((_W, D), jnp.float32) for _ in range(_NBUF)],
            [pltpu.SemaphoreType.DMA for _ in range(_NBUF)],
            [pltpu.SemaphoreType.DMA for _ in range(_NBUF)],
        ],
    )
    def emb(w_hbm, idx_hbm, out_hbm, idx_v, rows_v, gsem, ssem):
        wid = lax.axis_index("s") * num_cores + lax.axis_index("c")
        t0 = wid * tpw

        def fire(g, ib):
            pltpu.sync_copy(idx_hbm.at[pl.ds(t0 + g * _W, _W)], idx_v[ib])
            pltpu.async_copy(w_hbm.at[idx_v[ib]], rows_v[ib], gsem[ib])

        def wait_gather(ib):
            pltpu.make_async_copy(
                w_hbm.at[idx_v[ib]], rows_v[ib], gsem[ib]).wait()

        def store(g, ib):
            pltpu.async_copy(
                rows_v[ib], out_hbm.at[pl.ds(t0 + g * _W, _W)], ssem[ib])

        def wait_store(ib):
            pltpu.make_async_copy(
                rows_v[ib], out_hbm.at[pl.ds(t0, _W)], ssem[ib]).wait()

        for b in range(_NBUF - 1):
            fire(b, b)

        def body(r, carry):
            for b in range(_NBUF):
                g = r * _NBUF + b
                wait_gather(b)
                store(g, b)
                ibf = (b - 1) % _NBUF
                gf = g + _NBUF - 1

                @pl.when(gf < wins)
                def _():
                    @pl.when(gf >= _NBUF)
                    def _():
                        wait_store(ibf)

                    fire(gf, ibf)

            return carry

        lax.fori_loop(0, rounds, body, 0)
        for b in range(_NBUF):
            wait_store(b)

    return emb(weight, ids_flat)


def kernel(token_ids, weight):
    B, H = token_ids.shape
    D = weight.shape[1]
    out = _emb_lookup(weight, token_ids.astype(jnp.int32).reshape(B * H))
    return out.reshape(B, H, D)
